# baseline (speedup 1.0000x reference)
import functools

import jax
import jax.numpy as jnp
import numpy as np
from jax import lax
from jax.experimental import pallas as pl
from jax.experimental.pallas import tpu as pltpu
from jax.experimental.pallas import tpu_sc as plsc

N = 10000
E = 320000
HID = 128
POS = 16
NCLS = 10
NLAYER = 4
BN_EPS = 1e-5

NC = 2
NS = 16
NW = NC * NS

NPAD = 10240
CH = 128
CPT = 80
NH = 2
CPH = CPT // NH
EPAD = NW * CPT * CH
RPT = NPAD // NS
NBUF = 2
BR = 512

_mesh = functools.partial(
    plsc.VectorSubcoreMesh, core_axis_name="c", subcore_axis_name="s",
    num_cores=NC, num_subcores=NS)


def _zero_rows(rows):
    z = jnp.zeros((16,), jnp.float32)

    def body(i, _):
        r = i // (HID // 16)
        k = i % (HID // 16)
        rows[r, pl.ds(k * 16, 16)] = z
        return 0

    lax.fori_loop(0, CH * (HID // 16), body, 0)


def _deg_body(src_hbm, dst_hbm, dout_hbm, din_hbm, srcv, dstv, onesv, zbuf,
              dout_sh, din_sh, sem, fsem):
    c = lax.axis_index("c")
    s = lax.axis_index("s")
    wid = s * NC + c

    i1 = pltpu.async_copy(src_hbm.at[wid], srcv, sem)
    i2 = pltpu.async_copy(dst_hbm.at[wid], dstv, sem)

    one = jnp.full((16,), 1.0, jnp.float32)
    z = jnp.zeros((16,), jnp.float32)

    def fill_ones(i, _):
        onesv[pl.ds(i * 16, 16)] = one
        return 0

    lax.fori_loop(0, CH // 16, fill_ones, 0)

    def fill_z(i, _):
        zbuf[pl.ds(i * 16, 16)] = z
        return 0

    lax.fori_loop(0, RPT // 16, fill_z, 0)

    pltpu.sync_copy(zbuf, dout_sh.at[pl.ds(s * RPT, RPT)])
    pltpu.sync_copy(zbuf, din_sh.at[pl.ds(s * RPT, RPT)])
    plsc.subcore_barrier()
    i1.wait()
    i2.wait()

    def add_src(j, _):
        pltpu.async_copy(onesv, dout_sh.at[srcv.at[j]], fsem, add=True)
        return 0

    lax.fori_loop(0, CPT, add_src, 0)

    def add_dst(j, _):
        pltpu.async_copy(onesv, din_sh.at[dstv.at[j]], fsem, add=True)
        return 0

    lax.fori_loop(0, CPT, add_dst, 0)

    def drain(j, _):
        pltpu.make_async_copy(onesv, dout_sh.at[srcv.at[0]], fsem).wait()
        return 0

    lax.fori_loop(0, 2 * CPT, drain, 0)

    plsc.subcore_barrier()

    @pl.when(s == 0)
    def _():
        pltpu.sync_copy(dout_sh, dout_hbm.at[pl.ds(c * NPAD, NPAD)])
        pltpu.sync_copy(din_sh, din_hbm.at[pl.ds(c * NPAD, NPAD)])


def _sc_degrees(srcp, dstp):
    call = pl.kernel(
        _deg_body,
        out_type=[jax.ShapeDtypeStruct((NC * NPAD,), jnp.float32),
                  jax.ShapeDtypeStruct((NC * NPAD,), jnp.float32)],
        mesh=_mesh(),
        scratch_types=[
            pltpu.VMEM((CPT, CH), jnp.int32),
            pltpu.VMEM((CPT, CH), jnp.int32),
            pltpu.VMEM((CH,), jnp.float32),
            pltpu.VMEM((RPT,), jnp.float32),
            pltpu.VMEM_SHARED((NPAD,), jnp.float32),
            pltpu.VMEM_SHARED((NPAD,), jnp.float32),
            pltpu.SemaphoreType.DMA,
            pltpu.SemaphoreType.DMA,
        ],
    )
    return call(srcp, dstp)


def _mp_body(src_hbm, dst_hbm, hs_hbm, out_hbm, srcv, dstv, rows, agg_sh,
             sem, gsem, ssem, osem):
    c = lax.axis_index("c")
    s = lax.axis_index("s")
    wid = s * NC + c

    i1 = pltpu.async_copy(src_hbm.at[wid].at[0], srcv, sem)
    i2 = pltpu.async_copy(dst_hbm.at[wid].at[0], dstv, sem)
    _zero_rows(rows.at[0])
    zcs = [pltpu.async_copy(rows.at[0],
                            agg_sh.at[pl.ds(s * RPT + k * CH, CH)], osem)
           for k in range(RPT // CH)]
    for zc in zcs:
        zc.wait()
    plsc.subcore_barrier()
    i1.wait()
    i2.wait()

    def wait_g(b):
        pltpu.make_async_copy(hs_hbm.at[srcv.at[0]], rows.at[b],
                              gsem.at[b]).wait()

    def wait_s(b):
        pltpu.make_async_copy(rows.at[b], agg_sh.at[dstv.at[0]],
                              ssem.at[b]).wait()

    for h in range(NH):
        if h > 0:
            pltpu.async_copy(src_hbm.at[wid].at[h], srcv, sem).wait()
            pltpu.async_copy(dst_hbm.at[wid].at[h], dstv, sem).wait()

        pltpu.async_copy(hs_hbm.at[srcv.at[0]], rows.at[0], gsem.at[0])

        def rstep(r, _):
            j = r * 2

            @pl.when(r >= 1)
            def _():
                wait_s(1)

            pltpu.async_copy(hs_hbm.at[srcv.at[j + 1]], rows.at[1],
                             gsem.at[1])
            wait_g(0)
            pltpu.async_copy(rows.at[0], agg_sh.at[dstv.at[j]], ssem.at[0],
                             add=True)
            wait_s(0)

            @pl.when(r < CPH // 2 - 1)
            def _():
                pltpu.async_copy(hs_hbm.at[srcv.at[j + 2]], rows.at[0],
                                 gsem.at[0])

            wait_g(1)
            pltpu.async_copy(rows.at[1], agg_sh.at[dstv.at[j + 1]],
                             ssem.at[1], add=True)
            return 0

        lax.fori_loop(0, CPH // 2, rstep, 0)
        wait_s(1)

    plsc.subcore_barrier()

    ocs = [pltpu.async_copy(agg_sh.at[pl.ds(s * RPT + k * CH, CH)],
                            out_hbm.at[pl.ds(c * NPAD + s * RPT + k * CH,
                                             CH)], osem)
           for k in range(RPT // CH)]
    for oc in ocs:
        oc.wait()


def _sc_message_pass(srcp, dstp, hs):
    call = pl.kernel(
        _mp_body,
        out_type=jax.ShapeDtypeStruct((NC * NPAD, HID), jnp.float32),
        mesh=_mesh(),
        scratch_types=[
            pltpu.VMEM((CPH, CH), jnp.int32),
            pltpu.VMEM((CPH, CH), jnp.int32),
            pltpu.VMEM((NBUF, CH, HID), jnp.float32),
            pltpu.VMEM_SHARED((NPAD, HID), jnp.float32),
            pltpu.SemaphoreType.DMA,
            pltpu.SemaphoreType.DMA((NBUF,)),
            pltpu.SemaphoreType.DMA((NBUF,)),
            pltpu.SemaphoreType.DMA,
        ],
    )
    return call(srcp.reshape(NW, NH, CPH, CH), dstp.reshape(NW, NH, CPH, CH),
                hs)


def _prep_body(pos_ref, wpos_ref, bpos_ref, dout_ref, din_ref,
               hf_ref, hs_ref, ns_ref, nd_ref):
    hf = jnp.dot(pos_ref[...], wpos_ref[...],
                 preferred_element_type=jnp.float32) + bpos_ref[...]
    do = dout_ref[0] + dout_ref[1]
    di = din_ref[0] + din_ref[1]
    ns = jnp.where(do > 0, lax.rsqrt(jnp.maximum(do, 1.0)), 0.0)
    nd = jnp.where(di > 0, lax.rsqrt(jnp.maximum(di, 1.0)), 0.0)
    hf_ref[...] = hf
    hs_ref[...] = hf * ns
    ns_ref[...] = ns
    nd_ref[...] = nd


def _tc_prep(posp, wpos, bpos, dout, din):
    grid = (NPAD // BR,)
    return pl.pallas_call(
        _prep_body,
        grid=grid,
        in_specs=[
            pl.BlockSpec((BR, POS), lambda i: (i, 0)),
            pl.BlockSpec((POS, HID), lambda i: (0, 0)),
            pl.BlockSpec((1, HID), lambda i: (0, 0)),
            pl.BlockSpec((NC, BR, 1), lambda i: (0, i, 0)),
            pl.BlockSpec((NC, BR, 1), lambda i: (0, i, 0)),
        ],
        out_specs=[
            pl.BlockSpec((BR, HID), lambda i: (i, 0)),
            pl.BlockSpec((BR, HID), lambda i: (i, 0)),
            pl.BlockSpec((BR, 1), lambda i: (i, 0)),
            pl.BlockSpec((BR, 1), lambda i: (i, 0)),
        ],
        out_shape=[
            jax.ShapeDtypeStruct((NPAD, HID), jnp.float32),
            jax.ShapeDtypeStruct((NPAD, HID), jnp.float32),
            jax.ShapeDtypeStruct((NPAD, 1), jnp.float32),
            jax.ShapeDtypeStruct((NPAD, 1), jnp.float32),
        ],
    )(posp, wpos, bpos, dout, din)


def _layer_body(agg_ref, nd_ref, ns_ref, hf_ref, w_ref, b_ref,
                hf_out, hs_out):
    agg = (agg_ref[0] + agg_ref[1]) * nd_ref[...]
    out = jnp.dot(agg, w_ref[...], preferred_element_type=jnp.float32)
    out = jnp.maximum(out + b_ref[...], 0.0)
    hfn = hf_ref[...] + out
    hf_out[...] = hfn
    hs_out[...] = hfn * ns_ref[...]


def _tc_layer(aggp, nd, ns, hf, w, b):
    grid = (NPAD // BR,)
    return pl.pallas_call(
        _layer_body,
        grid=grid,
        in_specs=[
            pl.BlockSpec((NC, BR, HID), lambda i: (0, i, 0)),
            pl.BlockSpec((BR, 1), lambda i: (i, 0)),
            pl.BlockSpec((BR, 1), lambda i: (i, 0)),
            pl.BlockSpec((BR, HID), lambda i: (i, 0)),
            pl.BlockSpec((HID, HID), lambda i: (0, 0)),
            pl.BlockSpec((1, HID), lambda i: (0, 0)),
        ],
        out_specs=[
            pl.BlockSpec((BR, HID), lambda i: (i, 0)),
            pl.BlockSpec((BR, HID), lambda i: (i, 0)),
        ],
        out_shape=[
            jax.ShapeDtypeStruct((NPAD, HID), jnp.float32),
            jax.ShapeDtypeStruct((NPAD, HID), jnp.float32),
        ],
    )(aggp, nd, ns, hf, w, b)


def _layer4_body(agg_ref, nd_ref, hf_ref, w_ref, b_ref, w0_ref, b0_ref,
                 w1_ref, b1_ref, w2_ref, b2_ref, y_ref, acc_ref):
    i = pl.program_id(0)
    agg = (agg_ref[0] + agg_ref[1]) * nd_ref[...]
    out = jnp.dot(agg, w_ref[...], preferred_element_type=jnp.float32)
    out = jnp.maximum(out + b_ref[...], 0.0)
    hfn = hf_ref[...] + out
    rid = lax.broadcasted_iota(jnp.int32, (BR, 1), 0) + i * BR
    bsum = jnp.sum(jnp.where(rid < N, hfn, 0.0), axis=0, keepdims=True)

    @pl.when(i == 0)
    def _():
        acc_ref[...] = bsum

    @pl.when(i > 0)
    def _():
        acc_ref[...] += bsum

    @pl.when(i == NPAD // BR - 1)
    def _():
        hg = acc_ref[...] * (1.0 / N)
        y = jnp.maximum(jnp.dot(hg, w0_ref[...],
                                preferred_element_type=jnp.float32)
                        + b0_ref[...], 0.0)
        y = jnp.maximum(jnp.dot(y, w1_ref[...],
                                preferred_element_type=jnp.float32)
                        + b1_ref[...], 0.0)
        y = jnp.dot(y, w2_ref[...], preferred_element_type=jnp.float32) \
            + b2_ref[...]
        y_ref[...] = y


def _tc_layer4(aggp, nd, hf, w, b, w0, b0, w1, b1, w2, b2):
    grid = (NPAD // BR,)
    const = pl.BlockSpec((1, HID), lambda i: (0, 0))
    mat = pl.BlockSpec((HID, HID), lambda i: (0, 0))
    return pl.pallas_call(
        _layer4_body,
        grid=grid,
        in_specs=[
            pl.BlockSpec((NC, BR, HID), lambda i: (0, i, 0)),
            pl.BlockSpec((BR, 1), lambda i: (i, 0)),
            pl.BlockSpec((BR, HID), lambda i: (i, 0)),
            mat, const, mat, const, mat, const, mat, const,
        ],
        out_specs=pl.BlockSpec((1, HID), lambda i: (0, 0)),
        out_shape=jax.ShapeDtypeStruct((1, HID), jnp.float32),
        scratch_shapes=[pltpu.VMEM((1, HID), jnp.float32)],
    )(aggp, nd, hf, w, b, w0, b0, w1, b1, w2, b2)


def _pad_mat(w, rows, cols):
    out = jnp.zeros((rows, cols), w.dtype)
    return out.at[: w.shape[0], : w.shape[1]].set(w)


def kernel(edge_index, h, e, pos_enc, params):
    del h, e
    src = edge_index[0]
    dst = edge_index[1]

    npad_e = EPAD - E
    pad_idx = N + (jnp.arange(npad_e, dtype=jnp.int32) % (NPAD - N))
    srcp = jnp.concatenate([src, pad_idx]).reshape(NW, CPT, CH)
    dstp = jnp.concatenate([dst, pad_idx]).reshape(NW, CPT, CH)

    posp = jnp.zeros((NPAD, POS), jnp.float32).at[:N].set(pos_enc)

    dout, din = _sc_degrees(srcp, dstp)
    dout = dout.reshape(NC, NPAD, 1)
    din = din.reshape(NC, NPAD, 1)

    hf, hs, ns, nd = _tc_prep(posp, params["W_pos"],
                              params["b_pos"].reshape(1, HID), dout, din)

    inv_bn = 1.0 / np.sqrt(1.0 + BN_EPS)

    def fold(l):
        g = params["gamma"][l] * inv_bn
        w = params["W"][l] * g[None, :]
        b = (params["b"][l] * g + params["beta"][l]).reshape(1, HID)
        return w, b

    for l in range(NLAYER - 1):
        w, b = fold(l)
        aggp = _sc_message_pass(srcp, dstp, hs).reshape(NC, NPAD, HID)
        hf, hs = _tc_layer(aggp, nd, ns, hf, w, b)

    w0 = _pad_mat(params["Wr"][0], HID, HID)
    b0 = _pad_mat(params["br"][0].reshape(1, -1), 1, HID)
    w1 = _pad_mat(params["Wr"][1], HID, HID)
    b1 = _pad_mat(params["br"][1].reshape(1, -1), 1, HID)
    w2 = _pad_mat(params["Wr"][2], HID, HID)
    b2 = _pad_mat(params["br"][2].reshape(1, -1), 1, HID)

    w, b = fold(NLAYER - 1)
    aggp = _sc_message_pass(srcp, dstp, hs).reshape(NC, NPAD, HID)
    y = _tc_layer4(aggp, nd, hf, w, b, w0, b0, w1, b1, w2, b2)
    return y[:, :NCLS]


# speedup vs baseline: 1.0014x; 1.0014x over previous
import functools

import jax
import jax.numpy as jnp
import numpy as np
from jax import lax
from jax.experimental import pallas as pl
from jax.experimental.pallas import tpu as pltpu
from jax.experimental.pallas import tpu_sc as plsc

N = 10000
E = 320000
HID = 128
POS = 16
NCLS = 10
NLAYER = 4
BN_EPS = 1e-5

NC = 2
NS = 16
NW = NC * NS

NPAD = 10240
CH = 128
CPT = 80
NH = 2
CPH = CPT // NH
EPAD = NW * CPT * CH
RPT = NPAD // NS
NBUF = 2
BR = 512

_mesh = functools.partial(
    plsc.VectorSubcoreMesh, core_axis_name="c", subcore_axis_name="s",
    num_cores=NC, num_subcores=NS)


def _zero_rows(rows):
    z = jnp.zeros((16,), jnp.float32)

    def body(i, _):
        r = i // (HID // 16)
        k = i % (HID // 16)
        rows[r, pl.ds(k * 16, 16)] = z
        return 0

    lax.fori_loop(0, CH * (HID // 16), body, 0)


def _deg_body(src_hbm, dst_hbm, dout_hbm, din_hbm, srcv, dstv, onesv, zbuf,
              dout_sh, din_sh, sem, fsem):
    c = lax.axis_index("c")
    s = lax.axis_index("s")
    wid = s * NC + c

    i1 = pltpu.async_copy(src_hbm.at[wid], srcv, sem)
    i2 = pltpu.async_copy(dst_hbm.at[wid], dstv, sem)

    one = jnp.full((16,), 1.0, jnp.float32)
    z = jnp.zeros((16,), jnp.float32)

    def fill_ones(i, _):
        onesv[pl.ds(i * 16, 16)] = one
        return 0

    lax.fori_loop(0, CH // 16, fill_ones, 0)

    def fill_z(i, _):
        zbuf[pl.ds(i * 16, 16)] = z
        return 0

    lax.fori_loop(0, RPT // 16, fill_z, 0)

    pltpu.sync_copy(zbuf, dout_sh.at[pl.ds(s * RPT, RPT)])
    pltpu.sync_copy(zbuf, din_sh.at[pl.ds(s * RPT, RPT)])
    i1.wait()
    i2.wait()

    def add_src(j, _):
        pltpu.async_copy(onesv, dout_sh.at[srcv.at[j]], fsem, add=True)
        return 0

    lax.fori_loop(0, CPT, add_src, 0)

    def add_dst(j, _):
        pltpu.async_copy(onesv, din_sh.at[dstv.at[j]], fsem, add=True)
        return 0

    lax.fori_loop(0, CPT, add_dst, 0)

    def drain(j, _):
        pltpu.make_async_copy(onesv, dout_sh.at[srcv.at[0]], fsem).wait()
        return 0

    lax.fori_loop(0, 2 * CPT, drain, 0)

    plsc.subcore_barrier()

    @pl.when(s == 0)
    def _():
        pltpu.sync_copy(dout_sh, dout_hbm.at[pl.ds(c * NPAD, NPAD)])
        pltpu.sync_copy(din_sh, din_hbm.at[pl.ds(c * NPAD, NPAD)])


def _sc_degrees(srcp, dstp):
    call = pl.kernel(
        _deg_body,
        out_type=[jax.ShapeDtypeStruct((NC * NPAD,), jnp.float32),
                  jax.ShapeDtypeStruct((NC * NPAD,), jnp.float32)],
        mesh=_mesh(),
        scratch_types=[
            pltpu.VMEM((CPT, CH), jnp.int32),
            pltpu.VMEM((CPT, CH), jnp.int32),
            pltpu.VMEM((CH,), jnp.float32),
            pltpu.VMEM((RPT,), jnp.float32),
            pltpu.VMEM_SHARED((NPAD,), jnp.float32),
            pltpu.VMEM_SHARED((NPAD,), jnp.float32),
            pltpu.SemaphoreType.DMA,
            pltpu.SemaphoreType.DMA,
        ],
    )
    return call(srcp, dstp)


def _mp_body(src_hbm, dst_hbm, hs_hbm, out_hbm, srcv, dstv, rows, agg_sh,
             sem, gsem, ssem):
    c = lax.axis_index("c")
    s = lax.axis_index("s")
    wid = s * NC + c

    i1 = pltpu.async_copy(src_hbm.at[wid].at[0], srcv, sem)
    i2 = pltpu.async_copy(dst_hbm.at[wid].at[0], dstv, sem)
    _zero_rows(rows.at[0])
    for k in range(RPT // CH):
        pltpu.sync_copy(rows.at[0], agg_sh.at[pl.ds(s * RPT + k * CH, CH)])
    i1.wait()
    i2.wait()

    def wait_g(b):
        pltpu.make_async_copy(hs_hbm.at[srcv.at[0]], rows.at[b],
                              gsem.at[b]).wait()

    def wait_s(b):
        pltpu.make_async_copy(rows.at[b], agg_sh.at[dstv.at[0]],
                              ssem.at[b]).wait()

    for h in range(NH):
        if h > 0:
            pltpu.async_copy(src_hbm.at[wid].at[h], srcv, sem).wait()
            pltpu.async_copy(dst_hbm.at[wid].at[h], dstv, sem).wait()

        pltpu.async_copy(hs_hbm.at[srcv.at[0]], rows.at[0], gsem.at[0])
        if h == 0:
            plsc.subcore_barrier()

        def rstep(r, _):
            j = r * 2

            @pl.when(r >= 1)
            def _():
                wait_s(1)

            pltpu.async_copy(hs_hbm.at[srcv.at[j + 1]], rows.at[1],
                             gsem.at[1])
            wait_g(0)
            pltpu.async_copy(rows.at[0], agg_sh.at[dstv.at[j]], ssem.at[0],
                             add=True)
            wait_s(0)

            @pl.when(r < CPH // 2 - 1)
            def _():
                pltpu.async_copy(hs_hbm.at[srcv.at[j + 2]], rows.at[0],
                                 gsem.at[0])

            wait_g(1)
            pltpu.async_copy(rows.at[1], agg_sh.at[dstv.at[j + 1]],
                             ssem.at[1], add=True)
            return 0

        lax.fori_loop(0, CPH // 2, rstep, 0)
        wait_s(1)

    plsc.subcore_barrier()

    for k in range(RPT // CH):
        off = s * RPT + k * CH
        pltpu.sync_copy(agg_sh.at[pl.ds(off, CH)],
                        out_hbm.at[pl.ds(c * NPAD + off, CH)])


def _sc_message_pass(srcp, dstp, hs):
    call = pl.kernel(
        _mp_body,
        out_type=jax.ShapeDtypeStruct((NC * NPAD, HID), jnp.float32),
        mesh=_mesh(),
        scratch_types=[
            pltpu.VMEM((CPH, CH), jnp.int32),
            pltpu.VMEM((CPH, CH), jnp.int32),
            pltpu.VMEM((NBUF, CH, HID), jnp.float32),
            pltpu.VMEM_SHARED((NPAD, HID), jnp.float32),
            pltpu.SemaphoreType.DMA,
            pltpu.SemaphoreType.DMA((NBUF,)),
            pltpu.SemaphoreType.DMA((NBUF,)),
        ],
    )
    return call(srcp.reshape(NW, NH, CPH, CH), dstp.reshape(NW, NH, CPH, CH),
                hs)


def _prep_body(pos_ref, wpos_ref, bpos_ref, dout_ref, din_ref,
               hf_ref, hs_ref, ns_ref, nd_ref):
    hf = jnp.dot(pos_ref[...], wpos_ref[...],
                 preferred_element_type=jnp.float32) + bpos_ref[...]
    do = dout_ref[0] + dout_ref[1]
    di = din_ref[0] + din_ref[1]
    ns = jnp.where(do > 0, lax.rsqrt(jnp.maximum(do, 1.0)), 0.0)
    nd = jnp.where(di > 0, lax.rsqrt(jnp.maximum(di, 1.0)), 0.0)
    hf_ref[...] = hf
    hs_ref[...] = hf * ns
    ns_ref[...] = ns
    nd_ref[...] = nd


def _tc_prep(posp, wpos, bpos, dout, din):
    grid = (NPAD // BR,)
    return pl.pallas_call(
        _prep_body,
        grid=grid,
        in_specs=[
            pl.BlockSpec((BR, POS), lambda i: (i, 0)),
            pl.BlockSpec((POS, HID), lambda i: (0, 0)),
            pl.BlockSpec((1, HID), lambda i: (0, 0)),
            pl.BlockSpec((NC, BR, 1), lambda i: (0, i, 0)),
            pl.BlockSpec((NC, BR, 1), lambda i: (0, i, 0)),
        ],
        out_specs=[
            pl.BlockSpec((BR, HID), lambda i: (i, 0)),
            pl.BlockSpec((BR, HID), lambda i: (i, 0)),
            pl.BlockSpec((BR, 1), lambda i: (i, 0)),
            pl.BlockSpec((BR, 1), lambda i: (i, 0)),
        ],
        out_shape=[
            jax.ShapeDtypeStruct((NPAD, HID), jnp.float32),
            jax.ShapeDtypeStruct((NPAD, HID), jnp.float32),
            jax.ShapeDtypeStruct((NPAD, 1), jnp.float32),
            jax.ShapeDtypeStruct((NPAD, 1), jnp.float32),
        ],
    )(posp, wpos, bpos, dout, din)


def _layer_body(agg_ref, nd_ref, ns_ref, hf_ref, w_ref, b_ref,
                hf_out, hs_out):
    agg = (agg_ref[0] + agg_ref[1]) * nd_ref[...]
    out = jnp.dot(agg, w_ref[...], preferred_element_type=jnp.float32)
    out = jnp.maximum(out + b_ref[...], 0.0)
    hfn = hf_ref[...] + out
    hf_out[...] = hfn
    hs_out[...] = hfn * ns_ref[...]


def _tc_layer(aggp, nd, ns, hf, w, b):
    grid = (NPAD // BR,)
    return pl.pallas_call(
        _layer_body,
        grid=grid,
        in_specs=[
            pl.BlockSpec((NC, BR, HID), lambda i: (0, i, 0)),
            pl.BlockSpec((BR, 1), lambda i: (i, 0)),
            pl.BlockSpec((BR, 1), lambda i: (i, 0)),
            pl.BlockSpec((BR, HID), lambda i: (i, 0)),
            pl.BlockSpec((HID, HID), lambda i: (0, 0)),
            pl.BlockSpec((1, HID), lambda i: (0, 0)),
        ],
        out_specs=[
            pl.BlockSpec((BR, HID), lambda i: (i, 0)),
            pl.BlockSpec((BR, HID), lambda i: (i, 0)),
        ],
        out_shape=[
            jax.ShapeDtypeStruct((NPAD, HID), jnp.float32),
            jax.ShapeDtypeStruct((NPAD, HID), jnp.float32),
        ],
    )(aggp, nd, ns, hf, w, b)


def _layer4_body(agg_ref, nd_ref, hf_ref, w_ref, b_ref, w0_ref, b0_ref,
                 w1_ref, b1_ref, w2_ref, b2_ref, y_ref, acc_ref):
    i = pl.program_id(0)
    agg = (agg_ref[0] + agg_ref[1]) * nd_ref[...]
    out = jnp.dot(agg, w_ref[...], preferred_element_type=jnp.float32)
    out = jnp.maximum(out + b_ref[...], 0.0)
    hfn = hf_ref[...] + out
    rid = lax.broadcasted_iota(jnp.int32, (BR, 1), 0) + i * BR
    bsum = jnp.sum(jnp.where(rid < N, hfn, 0.0), axis=0, keepdims=True)

    @pl.when(i == 0)
    def _():
        acc_ref[...] = bsum

    @pl.when(i > 0)
    def _():
        acc_ref[...] += bsum

    @pl.when(i == NPAD // BR - 1)
    def _():
        hg = acc_ref[...] * (1.0 / N)
        y = jnp.maximum(jnp.dot(hg, w0_ref[...],
                                preferred_element_type=jnp.float32)
                        + b0_ref[...], 0.0)
        y = jnp.maximum(jnp.dot(y, w1_ref[...],
                                preferred_element_type=jnp.float32)
                        + b1_ref[...], 0.0)
        y = jnp.dot(y, w2_ref[...], preferred_element_type=jnp.float32) \
            + b2_ref[...]
        y_ref[...] = y


def _tc_layer4(aggp, nd, hf, w, b, w0, b0, w1, b1, w2, b2):
    grid = (NPAD // BR,)
    const = pl.BlockSpec((1, HID), lambda i: (0, 0))
    mat = pl.BlockSpec((HID, HID), lambda i: (0, 0))
    return pl.pallas_call(
        _layer4_body,
        grid=grid,
        in_specs=[
            pl.BlockSpec((NC, BR, HID), lambda i: (0, i, 0)),
            pl.BlockSpec((BR, 1), lambda i: (i, 0)),
            pl.BlockSpec((BR, HID), lambda i: (i, 0)),
            mat, const, mat, const, mat, const, mat, const,
        ],
        out_specs=pl.BlockSpec((1, HID), lambda i: (0, 0)),
        out_shape=jax.ShapeDtypeStruct((1, HID), jnp.float32),
        scratch_shapes=[pltpu.VMEM((1, HID), jnp.float32)],
    )(aggp, nd, hf, w, b, w0, b0, w1, b1, w2, b2)


def _pad_mat(w, rows, cols):
    out = jnp.zeros((rows, cols), w.dtype)
    return out.at[: w.shape[0], : w.shape[1]].set(w)


def kernel(edge_index, h, e, pos_enc, params):
    del h, e
    src = edge_index[0]
    dst = edge_index[1]

    npad_e = EPAD - E
    pad_idx = N + (jnp.arange(npad_e, dtype=jnp.int32) % (NPAD - N))
    srcp = jnp.concatenate([src, pad_idx]).reshape(NW, CPT, CH)
    dstp = jnp.concatenate([dst, pad_idx]).reshape(NW, CPT, CH)

    posp = jnp.zeros((NPAD, POS), jnp.float32).at[:N].set(pos_enc)

    dout, din = _sc_degrees(srcp, dstp)
    dout = dout.reshape(NC, NPAD, 1)
    din = din.reshape(NC, NPAD, 1)

    hf, hs, ns, nd = _tc_prep(posp, params["W_pos"],
                              params["b_pos"].reshape(1, HID), dout, din)

    inv_bn = 1.0 / np.sqrt(1.0 + BN_EPS)

    def fold(l):
        g = params["gamma"][l] * inv_bn
        w = params["W"][l] * g[None, :]
        b = (params["b"][l] * g + params["beta"][l]).reshape(1, HID)
        return w, b

    for l in range(NLAYER - 1):
        w, b = fold(l)
        aggp = _sc_message_pass(srcp, dstp, hs).reshape(NC, NPAD, HID)
        hf, hs = _tc_layer(aggp, nd, ns, hf, w, b)

    w0 = _pad_mat(params["Wr"][0], HID, HID)
    b0 = _pad_mat(params["br"][0].reshape(1, -1), 1, HID)
    w1 = _pad_mat(params["Wr"][1], HID, HID)
    b1 = _pad_mat(params["br"][1].reshape(1, -1), 1, HID)
    w2 = _pad_mat(params["Wr"][2], HID, HID)
    b2 = _pad_mat(params["br"][2].reshape(1, -1), 1, HID)

    w, b = fold(NLAYER - 1)
    aggp = _sc_message_pass(srcp, dstp, hs).reshape(NC, NPAD, HID)
    y = _tc_layer4(aggp, nd, hf, w, b, w0, b0, w1, b1, w2, b2)
    return y[:, :NCLS]


# speedup vs baseline: 1.0538x; 1.0523x over previous
import functools

import jax
import jax.numpy as jnp
import numpy as np
from jax import lax
from jax.experimental import pallas as pl
from jax.experimental.pallas import tpu as pltpu
from jax.experimental.pallas import tpu_sc as plsc

N = 10000
E = 320000
HID = 128
POS = 16
NCLS = 10
NLAYER = 4
BN_EPS = 1e-5

NC = 2
NS = 16
NW = NC * NS

NPAD = 10240
CH = 128
CPT = 80
NH = 2
CPH = CPT // NH
EPAD = NW * CPT * CH
RPT = NPAD // NS
NBUF = 2
BR = 1024

_mesh = functools.partial(
    plsc.VectorSubcoreMesh, core_axis_name="c", subcore_axis_name="s",
    num_cores=NC, num_subcores=NS)


def _zero_rows(rows):
    z = jnp.zeros((16,), jnp.float32)

    def body(i, _):
        r = i // (HID // 16)
        k = i % (HID // 16)
        rows[r, pl.ds(k * 16, 16)] = z
        return 0

    lax.fori_loop(0, CH * (HID // 16), body, 0)


def _deg_body(src_hbm, dst_hbm, dout_hbm, din_hbm, srcv, dstv, onesv, zbuf,
              dout_sh, din_sh, sem, fsem):
    c = lax.axis_index("c")
    s = lax.axis_index("s")
    wid = s * NC + c

    i1 = pltpu.async_copy(src_hbm.at[wid], srcv, sem)
    i2 = pltpu.async_copy(dst_hbm.at[wid], dstv, sem)

    one = jnp.full((16,), 1.0, jnp.float32)
    z = jnp.zeros((16,), jnp.float32)

    def fill_ones(i, _):
        onesv[pl.ds(i * 16, 16)] = one
        return 0

    lax.fori_loop(0, CH // 16, fill_ones, 0)

    def fill_z(i, _):
        zbuf[pl.ds(i * 16, 16)] = z
        return 0

    lax.fori_loop(0, RPT // 16, fill_z, 0)

    pltpu.sync_copy(zbuf, dout_sh.at[pl.ds(s * RPT, RPT)])
    pltpu.sync_copy(zbuf, din_sh.at[pl.ds(s * RPT, RPT)])
    plsc.subcore_barrier()
    i1.wait()
    i2.wait()

    def add_src(j, _):
        pltpu.async_copy(onesv, dout_sh.at[srcv.at[j]], fsem, add=True)
        return 0

    lax.fori_loop(0, CPT, add_src, 0)

    def add_dst(j, _):
        pltpu.async_copy(onesv, din_sh.at[dstv.at[j]], fsem, add=True)
        return 0

    lax.fori_loop(0, CPT, add_dst, 0)

    def drain(j, _):
        pltpu.make_async_copy(onesv, dout_sh.at[srcv.at[0]], fsem).wait()
        return 0

    lax.fori_loop(0, 2 * CPT, drain, 0)

    plsc.subcore_barrier()

    @pl.when(s == 0)
    def _():
        pltpu.sync_copy(dout_sh, dout_hbm.at[pl.ds(c * NPAD, NPAD)])
        pltpu.sync_copy(din_sh, din_hbm.at[pl.ds(c * NPAD, NPAD)])


def _sc_degrees(srcp, dstp):
    call = pl.kernel(
        _deg_body,
        out_type=[jax.ShapeDtypeStruct((NC * NPAD,), jnp.float32),
                  jax.ShapeDtypeStruct((NC * NPAD,), jnp.float32)],
        mesh=_mesh(),
        scratch_types=[
            pltpu.VMEM((CPT, CH), jnp.int32),
            pltpu.VMEM((CPT, CH), jnp.int32),
            pltpu.VMEM((CH,), jnp.float32),
            pltpu.VMEM((RPT,), jnp.float32),
            pltpu.VMEM_SHARED((NPAD,), jnp.float32),
            pltpu.VMEM_SHARED((NPAD,), jnp.float32),
            pltpu.SemaphoreType.DMA,
            pltpu.SemaphoreType.DMA,
        ],
    )
    return call(srcp, dstp)


def _mp_body(src_hbm, dst_hbm, hs_hbm, out_hbm, srcv, dstv, rows, agg_sh,
             sem, gsem, ssem):
    c = lax.axis_index("c")
    s = lax.axis_index("s")
    wid = s * NC + c

    i1 = pltpu.async_copy(src_hbm.at[wid].at[0], srcv, sem)
    i2 = pltpu.async_copy(dst_hbm.at[wid].at[0], dstv, sem)
    _zero_rows(rows.at[0])
    for k in range(RPT // CH):
        pltpu.sync_copy(rows.at[0], agg_sh.at[pl.ds(s * RPT + k * CH, CH)])
    plsc.subcore_barrier()
    i1.wait()
    i2.wait()

    def wait_g(b):
        pltpu.make_async_copy(hs_hbm.at[srcv.at[0]], rows.at[b],
                              gsem.at[b]).wait()

    def wait_s(b):
        pltpu.make_async_copy(rows.at[b], agg_sh.at[dstv.at[0]],
                              ssem.at[b]).wait()

    for h in range(NH):
        if h > 0:
            pltpu.async_copy(src_hbm.at[wid].at[h], srcv, sem).wait()
            pltpu.async_copy(dst_hbm.at[wid].at[h], dstv, sem).wait()

        pltpu.async_copy(hs_hbm.at[srcv.at[0]], rows.at[0], gsem.at[0])

        def rstep(r, _):
            j = r * 2

            @pl.when(r >= 1)
            def _():
                wait_s(1)

            pltpu.async_copy(hs_hbm.at[srcv.at[j + 1]], rows.at[1],
                             gsem.at[1])
            wait_g(0)
            pltpu.async_copy(rows.at[0], agg_sh.at[dstv.at[j]], ssem.at[0],
                             add=True)
            wait_s(0)

            @pl.when(r < CPH // 2 - 1)
            def _():
                pltpu.async_copy(hs_hbm.at[srcv.at[j + 2]], rows.at[0],
                                 gsem.at[0])

            wait_g(1)
            pltpu.async_copy(rows.at[1], agg_sh.at[dstv.at[j + 1]],
                             ssem.at[1], add=True)
            return 0

        lax.fori_loop(0, CPH // 2, rstep, 0)
        wait_s(1)

    plsc.subcore_barrier()

    for k in range(RPT // CH):
        off = s * RPT + k * CH
        pltpu.sync_copy(agg_sh.at[pl.ds(off, CH)],
                        out_hbm.at[pl.ds(c * NPAD + off, CH)])


def _sc_message_pass(srcp, dstp, hs):
    call = pl.kernel(
        _mp_body,
        out_type=jax.ShapeDtypeStruct((NC * NPAD, HID), jnp.float32),
        mesh=_mesh(),
        scratch_types=[
            pltpu.VMEM((CPH, CH), jnp.int32),
            pltpu.VMEM((CPH, CH), jnp.int32),
            pltpu.VMEM((NBUF, CH, HID), jnp.float32),
            pltpu.VMEM_SHARED((NPAD, HID), jnp.float32),
            pltpu.SemaphoreType.DMA,
            pltpu.SemaphoreType.DMA((NBUF,)),
            pltpu.SemaphoreType.DMA((NBUF,)),
        ],
    )
    return call(srcp.reshape(NW, NH, CPH, CH), dstp.reshape(NW, NH, CPH, CH),
                hs)


def _prep_body(pos_ref, wpos_ref, bpos_ref, dout_ref, din_ref,
               hf_ref, hs_ref, ns_ref, nd_ref):
    hf = jnp.dot(pos_ref[...], wpos_ref[...],
                 preferred_element_type=jnp.float32) + bpos_ref[...]
    do = dout_ref[0] + dout_ref[1]
    di = din_ref[0] + din_ref[1]
    ns = jnp.where(do > 0, lax.rsqrt(jnp.maximum(do, 1.0)), 0.0)
    nd = jnp.where(di > 0, lax.rsqrt(jnp.maximum(di, 1.0)), 0.0)
    hf_ref[...] = hf
    hs_ref[...] = hf * ns
    ns_ref[...] = ns
    nd_ref[...] = nd


def _tc_prep(posp, wpos, bpos, dout, din):
    grid = (NPAD // BR,)
    return pl.pallas_call(
        _prep_body,
        grid=grid,
        in_specs=[
            pl.BlockSpec((BR, POS), lambda i: (i, 0)),
            pl.BlockSpec((POS, HID), lambda i: (0, 0)),
            pl.BlockSpec((1, HID), lambda i: (0, 0)),
            pl.BlockSpec((NC, BR, 1), lambda i: (0, i, 0)),
            pl.BlockSpec((NC, BR, 1), lambda i: (0, i, 0)),
        ],
        out_specs=[
            pl.BlockSpec((BR, HID), lambda i: (i, 0)),
            pl.BlockSpec((BR, HID), lambda i: (i, 0)),
            pl.BlockSpec((BR, 1), lambda i: (i, 0)),
            pl.BlockSpec((BR, 1), lambda i: (i, 0)),
        ],
        out_shape=[
            jax.ShapeDtypeStruct((NPAD, HID), jnp.float32),
            jax.ShapeDtypeStruct((NPAD, HID), jnp.float32),
            jax.ShapeDtypeStruct((NPAD, 1), jnp.float32),
            jax.ShapeDtypeStruct((NPAD, 1), jnp.float32),
        ],
    )(posp, wpos, bpos, dout, din)


def _layer_body(agg_ref, nd_ref, ns_ref, hf_ref, w_ref, b_ref,
                hf_out, hs_out):
    agg = (agg_ref[0] + agg_ref[1]) * nd_ref[...]
    out = jnp.dot(agg, w_ref[...], preferred_element_type=jnp.float32)
    out = jnp.maximum(out + b_ref[...], 0.0)
    hfn = hf_ref[...] + out
    hf_out[...] = hfn
    hs_out[...] = hfn * ns_ref[...]


def _tc_layer(aggp, nd, ns, hf, w, b):
    grid = (NPAD // BR,)
    return pl.pallas_call(
        _layer_body,
        grid=grid,
        in_specs=[
            pl.BlockSpec((NC, BR, HID), lambda i: (0, i, 0)),
            pl.BlockSpec((BR, 1), lambda i: (i, 0)),
            pl.BlockSpec((BR, 1), lambda i: (i, 0)),
            pl.BlockSpec((BR, HID), lambda i: (i, 0)),
            pl.BlockSpec((HID, HID), lambda i: (0, 0)),
            pl.BlockSpec((1, HID), lambda i: (0, 0)),
        ],
        out_specs=[
            pl.BlockSpec((BR, HID), lambda i: (i, 0)),
            pl.BlockSpec((BR, HID), lambda i: (i, 0)),
        ],
        out_shape=[
            jax.ShapeDtypeStruct((NPAD, HID), jnp.float32),
            jax.ShapeDtypeStruct((NPAD, HID), jnp.float32),
        ],
    )(aggp, nd, ns, hf, w, b)


def _layer4_body(agg_ref, nd_ref, hf_ref, w_ref, b_ref, w0_ref, b0_ref,
                 w1_ref, b1_ref, w2_ref, b2_ref, y_ref, acc_ref):
    i = pl.program_id(0)
    agg = (agg_ref[0] + agg_ref[1]) * nd_ref[...]
    out = jnp.dot(agg, w_ref[...], preferred_element_type=jnp.float32)
    out = jnp.maximum(out + b_ref[...], 0.0)
    hfn = hf_ref[...] + out
    rid = lax.broadcasted_iota(jnp.int32, (BR, 1), 0) + i * BR
    bsum = jnp.sum(jnp.where(rid < N, hfn, 0.0), axis=0, keepdims=True)

    @pl.when(i == 0)
    def _():
        acc_ref[...] = bsum

    @pl.when(i > 0)
    def _():
        acc_ref[...] += bsum

    @pl.when(i == NPAD // BR - 1)
    def _():
        hg = acc_ref[...] * (1.0 / N)
        y = jnp.maximum(jnp.dot(hg, w0_ref[...],
                                preferred_element_type=jnp.float32)
                        + b0_ref[...], 0.0)
        y = jnp.maximum(jnp.dot(y, w1_ref[...],
                                preferred_element_type=jnp.float32)
                        + b1_ref[...], 0.0)
        y = jnp.dot(y, w2_ref[...], preferred_element_type=jnp.float32) \
            + b2_ref[...]
        y_ref[...] = y


def _tc_layer4(aggp, nd, hf, w, b, w0, b0, w1, b1, w2, b2):
    grid = (NPAD // BR,)
    const = pl.BlockSpec((1, HID), lambda i: (0, 0))
    mat = pl.BlockSpec((HID, HID), lambda i: (0, 0))
    return pl.pallas_call(
        _layer4_body,
        grid=grid,
        in_specs=[
            pl.BlockSpec((NC, BR, HID), lambda i: (0, i, 0)),
            pl.BlockSpec((BR, 1), lambda i: (i, 0)),
            pl.BlockSpec((BR, HID), lambda i: (i, 0)),
            mat, const, mat, const, mat, const, mat, const,
        ],
        out_specs=pl.BlockSpec((1, HID), lambda i: (0, 0)),
        out_shape=jax.ShapeDtypeStruct((1, HID), jnp.float32),
        scratch_shapes=[pltpu.VMEM((1, HID), jnp.float32)],
    )(aggp, nd, hf, w, b, w0, b0, w1, b1, w2, b2)


def _pad_mat(w, rows, cols):
    out = jnp.zeros((rows, cols), w.dtype)
    return out.at[: w.shape[0], : w.shape[1]].set(w)


def kernel(edge_index, h, e, pos_enc, params):
    del h, e
    src = edge_index[0]
    dst = edge_index[1]

    npad_e = EPAD - E
    pad_idx = N + (jnp.arange(npad_e, dtype=jnp.int32) % (NPAD - N))
    srcp = jnp.concatenate([src, pad_idx]).reshape(NW, CPT, CH)
    dstp = jnp.concatenate([dst, pad_idx]).reshape(NW, CPT, CH)

    posp = jnp.zeros((NPAD, POS), jnp.float32).at[:N].set(pos_enc)

    dout, din = _sc_degrees(srcp, dstp)
    dout = dout.reshape(NC, NPAD, 1)
    din = din.reshape(NC, NPAD, 1)

    hf, hs, ns, nd = _tc_prep(posp, params["W_pos"],
                              params["b_pos"].reshape(1, HID), dout, din)

    inv_bn = 1.0 / np.sqrt(1.0 + BN_EPS)

    def fold(l):
        g = params["gamma"][l] * inv_bn
        w = params["W"][l] * g[None, :]
        b = (params["b"][l] * g + params["beta"][l]).reshape(1, HID)
        return w, b

    for l in range(NLAYER - 1):
        w, b = fold(l)
        aggp = _sc_message_pass(srcp, dstp, hs).reshape(NC, NPAD, HID)
        hf, hs = _tc_layer(aggp, nd, ns, hf, w, b)

    w0 = _pad_mat(params["Wr"][0], HID, HID)
    b0 = _pad_mat(params["br"][0].reshape(1, -1), 1, HID)
    w1 = _pad_mat(params["Wr"][1], HID, HID)
    b1 = _pad_mat(params["br"][1].reshape(1, -1), 1, HID)
    w2 = _pad_mat(params["Wr"][2], HID, HID)
    b2 = _pad_mat(params["br"][2].reshape(1, -1), 1, HID)

    w, b = fold(NLAYER - 1)
    aggp = _sc_message_pass(srcp, dstp, hs).reshape(NC, NPAD, HID)
    y = _tc_layer4(aggp, nd, hf, w, b, w0, b0, w1, b1, w2, b2)
    return y[:, :NCLS]


# speedup vs baseline: 1.0730x; 1.0182x over previous
import functools

import jax
import jax.numpy as jnp
import numpy as np
from jax import lax
from jax.experimental import pallas as pl
from jax.experimental.pallas import tpu as pltpu
from jax.experimental.pallas import tpu_sc as plsc

N = 10000
E = 320000
HID = 128
POS = 16
NCLS = 10
NLAYER = 4
BN_EPS = 1e-5

NC = 2
NS = 16
NW = NC * NS

NPAD = 10240
CH = 128
CPT = 80
NH = 2
CPH = CPT // NH
EPAD = NW * CPT * CH
RPT = NPAD // NS
NBUF = 2
BR = 2048

_mesh = functools.partial(
    plsc.VectorSubcoreMesh, core_axis_name="c", subcore_axis_name="s",
    num_cores=NC, num_subcores=NS)


def _zero_rows(rows):
    z = jnp.zeros((16,), jnp.float32)

    def body(i, _):
        r = i // (HID // 16)
        k = i % (HID // 16)
        rows[r, pl.ds(k * 16, 16)] = z
        return 0

    lax.fori_loop(0, CH * (HID // 16), body, 0)


def _deg_body(src_hbm, dst_hbm, dout_hbm, din_hbm, srcv, dstv, onesv, zbuf,
              dout_sh, din_sh, sem, fsem):
    c = lax.axis_index("c")
    s = lax.axis_index("s")
    wid = s * NC + c

    i1 = pltpu.async_copy(src_hbm.at[wid], srcv, sem)
    i2 = pltpu.async_copy(dst_hbm.at[wid], dstv, sem)

    one = jnp.full((16,), 1.0, jnp.float32)
    z = jnp.zeros((16,), jnp.float32)

    def fill_ones(i, _):
        onesv[pl.ds(i * 16, 16)] = one
        return 0

    lax.fori_loop(0, CH // 16, fill_ones, 0)

    def fill_z(i, _):
        zbuf[pl.ds(i * 16, 16)] = z
        return 0

    lax.fori_loop(0, RPT // 16, fill_z, 0)

    pltpu.sync_copy(zbuf, dout_sh.at[pl.ds(s * RPT, RPT)])
    pltpu.sync_copy(zbuf, din_sh.at[pl.ds(s * RPT, RPT)])
    plsc.subcore_barrier()
    i1.wait()
    i2.wait()

    def add_src(j, _):
        pltpu.async_copy(onesv, dout_sh.at[srcv.at[j]], fsem, add=True)
        return 0

    lax.fori_loop(0, CPT, add_src, 0)

    def add_dst(j, _):
        pltpu.async_copy(onesv, din_sh.at[dstv.at[j]], fsem, add=True)
        return 0

    lax.fori_loop(0, CPT, add_dst, 0)

    def drain(j, _):
        pltpu.make_async_copy(onesv, dout_sh.at[srcv.at[0]], fsem).wait()
        return 0

    lax.fori_loop(0, 2 * CPT, drain, 0)

    plsc.subcore_barrier()

    @pl.when(s == 0)
    def _():
        pltpu.sync_copy(dout_sh, dout_hbm.at[pl.ds(c * NPAD, NPAD)])
        pltpu.sync_copy(din_sh, din_hbm.at[pl.ds(c * NPAD, NPAD)])


def _sc_degrees(srcp, dstp):
    call = pl.kernel(
        _deg_body,
        out_type=[jax.ShapeDtypeStruct((NC * NPAD,), jnp.float32),
                  jax.ShapeDtypeStruct((NC * NPAD,), jnp.float32)],
        mesh=_mesh(),
        scratch_types=[
            pltpu.VMEM((CPT, CH), jnp.int32),
            pltpu.VMEM((CPT, CH), jnp.int32),
            pltpu.VMEM((CH,), jnp.float32),
            pltpu.VMEM((RPT,), jnp.float32),
            pltpu.VMEM_SHARED((NPAD,), jnp.float32),
            pltpu.VMEM_SHARED((NPAD,), jnp.float32),
            pltpu.SemaphoreType.DMA,
            pltpu.SemaphoreType.DMA,
        ],
    )
    return call(srcp, dstp)


def _mp_body(src_hbm, dst_hbm, hs_hbm, out_hbm, srcv, dstv, rows, agg_sh,
             sem, gsem, ssem):
    c = lax.axis_index("c")
    s = lax.axis_index("s")
    wid = s * NC + c

    i1 = pltpu.async_copy(src_hbm.at[wid].at[0], srcv, sem)
    i2 = pltpu.async_copy(dst_hbm.at[wid].at[0], dstv, sem)
    _zero_rows(rows.at[0])
    for k in range(RPT // CH):
        pltpu.sync_copy(rows.at[0], agg_sh.at[pl.ds(s * RPT + k * CH, CH)])
    plsc.subcore_barrier()
    i1.wait()
    i2.wait()

    def wait_g(b):
        pltpu.make_async_copy(hs_hbm.at[srcv.at[0]], rows.at[b],
                              gsem.at[b]).wait()

    def wait_s(b):
        pltpu.make_async_copy(rows.at[b], agg_sh.at[dstv.at[0]],
                              ssem.at[b]).wait()

    for h in range(NH):
        if h > 0:
            pltpu.async_copy(src_hbm.at[wid].at[h], srcv, sem).wait()
            pltpu.async_copy(dst_hbm.at[wid].at[h], dstv, sem).wait()

        pltpu.async_copy(hs_hbm.at[srcv.at[0]], rows.at[0], gsem.at[0])

        def rstep(r, _):
            j = r * 2

            @pl.when(r >= 1)
            def _():
                wait_s(1)

            pltpu.async_copy(hs_hbm.at[srcv.at[j + 1]], rows.at[1],
                             gsem.at[1])
            wait_g(0)
            pltpu.async_copy(rows.at[0], agg_sh.at[dstv.at[j]], ssem.at[0],
                             add=True)
            wait_s(0)

            @pl.when(r < CPH // 2 - 1)
            def _():
                pltpu.async_copy(hs_hbm.at[srcv.at[j + 2]], rows.at[0],
                                 gsem.at[0])

            wait_g(1)
            pltpu.async_copy(rows.at[1], agg_sh.at[dstv.at[j + 1]],
                             ssem.at[1], add=True)
            return 0

        lax.fori_loop(0, CPH // 2, rstep, 0)
        wait_s(1)

    plsc.subcore_barrier()

    for k in range(RPT // CH):
        off = s * RPT + k * CH
        pltpu.sync_copy(agg_sh.at[pl.ds(off, CH)],
                        out_hbm.at[pl.ds(c * NPAD + off, CH)])


def _sc_message_pass(srcp, dstp, hs):
    call = pl.kernel(
        _mp_body,
        out_type=jax.ShapeDtypeStruct((NC * NPAD, HID), jnp.float32),
        mesh=_mesh(),
        scratch_types=[
            pltpu.VMEM((CPH, CH), jnp.int32),
            pltpu.VMEM((CPH, CH), jnp.int32),
            pltpu.VMEM((NBUF, CH, HID), jnp.float32),
            pltpu.VMEM_SHARED((NPAD, HID), jnp.float32),
            pltpu.SemaphoreType.DMA,
            pltpu.SemaphoreType.DMA((NBUF,)),
            pltpu.SemaphoreType.DMA((NBUF,)),
        ],
    )
    return call(srcp.reshape(NW, NH, CPH, CH), dstp.reshape(NW, NH, CPH, CH),
                hs)


def _prep_body(pos_ref, wpos_ref, bpos_ref, dout_ref, din_ref,
               hf_ref, hs_ref, ns_ref, nd_ref):
    hf = jnp.dot(pos_ref[...], wpos_ref[...],
                 preferred_element_type=jnp.float32) + bpos_ref[...]
    do = dout_ref[0] + dout_ref[1]
    di = din_ref[0] + din_ref[1]
    ns = jnp.where(do > 0, lax.rsqrt(jnp.maximum(do, 1.0)), 0.0)
    nd = jnp.where(di > 0, lax.rsqrt(jnp.maximum(di, 1.0)), 0.0)
    hf_ref[...] = hf
    hs_ref[...] = hf * ns
    ns_ref[...] = ns
    nd_ref[...] = nd


def _tc_prep(posp, wpos, bpos, dout, din):
    grid = (NPAD // BR,)
    return pl.pallas_call(
        _prep_body,
        grid=grid,
        in_specs=[
            pl.BlockSpec((BR, POS), lambda i: (i, 0)),
            pl.BlockSpec((POS, HID), lambda i: (0, 0)),
            pl.BlockSpec((1, HID), lambda i: (0, 0)),
            pl.BlockSpec((NC, BR, 1), lambda i: (0, i, 0)),
            pl.BlockSpec((NC, BR, 1), lambda i: (0, i, 0)),
        ],
        out_specs=[
            pl.BlockSpec((BR, HID), lambda i: (i, 0)),
            pl.BlockSpec((BR, HID), lambda i: (i, 0)),
            pl.BlockSpec((BR, 1), lambda i: (i, 0)),
            pl.BlockSpec((BR, 1), lambda i: (i, 0)),
        ],
        out_shape=[
            jax.ShapeDtypeStruct((NPAD, HID), jnp.float32),
            jax.ShapeDtypeStruct((NPAD, HID), jnp.float32),
            jax.ShapeDtypeStruct((NPAD, 1), jnp.float32),
            jax.ShapeDtypeStruct((NPAD, 1), jnp.float32),
        ],
    )(posp, wpos, bpos, dout, din)


def _layer_body(agg_ref, nd_ref, ns_ref, hf_ref, w_ref, b_ref,
                hf_out, hs_out):
    agg = (agg_ref[0] + agg_ref[1]) * nd_ref[...]
    out = jnp.dot(agg, w_ref[...], preferred_element_type=jnp.float32)
    out = jnp.maximum(out + b_ref[...], 0.0)
    hfn = hf_ref[...] + out
    hf_out[...] = hfn
    hs_out[...] = hfn * ns_ref[...]


def _tc_layer(aggp, nd, ns, hf, w, b):
    grid = (NPAD // BR,)
    return pl.pallas_call(
        _layer_body,
        grid=grid,
        in_specs=[
            pl.BlockSpec((NC, BR, HID), lambda i: (0, i, 0)),
            pl.BlockSpec((BR, 1), lambda i: (i, 0)),
            pl.BlockSpec((BR, 1), lambda i: (i, 0)),
            pl.BlockSpec((BR, HID), lambda i: (i, 0)),
            pl.BlockSpec((HID, HID), lambda i: (0, 0)),
            pl.BlockSpec((1, HID), lambda i: (0, 0)),
        ],
        out_specs=[
            pl.BlockSpec((BR, HID), lambda i: (i, 0)),
            pl.BlockSpec((BR, HID), lambda i: (i, 0)),
        ],
        out_shape=[
            jax.ShapeDtypeStruct((NPAD, HID), jnp.float32),
            jax.ShapeDtypeStruct((NPAD, HID), jnp.float32),
        ],
    )(aggp, nd, ns, hf, w, b)


def _layer4_body(agg_ref, nd_ref, hf_ref, w_ref, b_ref, w0_ref, b0_ref,
                 w1_ref, b1_ref, w2_ref, b2_ref, y_ref, acc_ref):
    i = pl.program_id(0)
    agg = (agg_ref[0] + agg_ref[1]) * nd_ref[...]
    out = jnp.dot(agg, w_ref[...], preferred_element_type=jnp.float32)
    out = jnp.maximum(out + b_ref[...], 0.0)
    hfn = hf_ref[...] + out
    rid = lax.broadcasted_iota(jnp.int32, (BR, 1), 0) + i * BR
    bsum = jnp.sum(jnp.where(rid < N, hfn, 0.0), axis=0, keepdims=True)

    @pl.when(i == 0)
    def _():
        acc_ref[...] = bsum

    @pl.when(i > 0)
    def _():
        acc_ref[...] += bsum

    @pl.when(i == NPAD // BR - 1)
    def _():
        hg = acc_ref[...] * (1.0 / N)
        y = jnp.maximum(jnp.dot(hg, w0_ref[...],
                                preferred_element_type=jnp.float32)
                        + b0_ref[...], 0.0)
        y = jnp.maximum(jnp.dot(y, w1_ref[...],
                                preferred_element_type=jnp.float32)
                        + b1_ref[...], 0.0)
        y = jnp.dot(y, w2_ref[...], preferred_element_type=jnp.float32) \
            + b2_ref[...]
        y_ref[...] = y


def _tc_layer4(aggp, nd, hf, w, b, w0, b0, w1, b1, w2, b2):
    grid = (NPAD // BR,)
    const = pl.BlockSpec((1, HID), lambda i: (0, 0))
    mat = pl.BlockSpec((HID, HID), lambda i: (0, 0))
    return pl.pallas_call(
        _layer4_body,
        grid=grid,
        in_specs=[
            pl.BlockSpec((NC, BR, HID), lambda i: (0, i, 0)),
            pl.BlockSpec((BR, 1), lambda i: (i, 0)),
            pl.BlockSpec((BR, HID), lambda i: (i, 0)),
            mat, const, mat, const, mat, const, mat, const,
        ],
        out_specs=pl.BlockSpec((1, HID), lambda i: (0, 0)),
        out_shape=jax.ShapeDtypeStruct((1, HID), jnp.float32),
        scratch_shapes=[pltpu.VMEM((1, HID), jnp.float32)],
    )(aggp, nd, hf, w, b, w0, b0, w1, b1, w2, b2)


def _pad_mat(w, rows, cols):
    out = jnp.zeros((rows, cols), w.dtype)
    return out.at[: w.shape[0], : w.shape[1]].set(w)


def kernel(edge_index, h, e, pos_enc, params):
    del h, e
    src = edge_index[0]
    dst = edge_index[1]

    npad_e = EPAD - E
    pad_idx = N + (jnp.arange(npad_e, dtype=jnp.int32) % (NPAD - N))
    srcp = jnp.concatenate([src, pad_idx]).reshape(NW, CPT, CH)
    dstp = jnp.concatenate([dst, pad_idx]).reshape(NW, CPT, CH)

    posp = jnp.zeros((NPAD, POS), jnp.float32).at[:N].set(pos_enc)

    dout, din = _sc_degrees(srcp, dstp)
    dout = dout.reshape(NC, NPAD, 1)
    din = din.reshape(NC, NPAD, 1)

    hf, hs, ns, nd = _tc_prep(posp, params["W_pos"],
                              params["b_pos"].reshape(1, HID), dout, din)

    inv_bn = 1.0 / np.sqrt(1.0 + BN_EPS)

    def fold(l):
        g = params["gamma"][l] * inv_bn
        w = params["W"][l] * g[None, :]
        b = (params["b"][l] * g + params["beta"][l]).reshape(1, HID)
        return w, b

    for l in range(NLAYER - 1):
        w, b = fold(l)
        aggp = _sc_message_pass(srcp, dstp, hs).reshape(NC, NPAD, HID)
        hf, hs = _tc_layer(aggp, nd, ns, hf, w, b)

    w0 = _pad_mat(params["Wr"][0], HID, HID)
    b0 = _pad_mat(params["br"][0].reshape(1, -1), 1, HID)
    w1 = _pad_mat(params["Wr"][1], HID, HID)
    b1 = _pad_mat(params["br"][1].reshape(1, -1), 1, HID)
    w2 = _pad_mat(params["Wr"][2], HID, HID)
    b2 = _pad_mat(params["br"][2].reshape(1, -1), 1, HID)

    w, b = fold(NLAYER - 1)
    aggp = _sc_message_pass(srcp, dstp, hs).reshape(NC, NPAD, HID)
    y = _tc_layer4(aggp, nd, hf, w, b, w0, b0, w1, b1, w2, b2)
    return y[:, :NCLS]


# speedup vs baseline: 1.0746x; 1.0016x over previous
import functools

import jax
import jax.numpy as jnp
import numpy as np
from jax import lax
from jax.experimental import pallas as pl
from jax.experimental.pallas import tpu as pltpu
from jax.experimental.pallas import tpu_sc as plsc

N = 10000
E = 320000
HID = 128
POS = 16
NCLS = 10
NLAYER = 4
BN_EPS = 1e-5

NC = 2
NS = 16
NW = NC * NS

NPAD = 10240
CH = 128
CPT = 80
NH = 2
CPH = CPT // NH
EPAD = NW * CPT * CH
RPT = NPAD // NS
NBUF = 2
BR = 5120

_mesh = functools.partial(
    plsc.VectorSubcoreMesh, core_axis_name="c", subcore_axis_name="s",
    num_cores=NC, num_subcores=NS)


def _zero_rows(rows):
    z = jnp.zeros((16,), jnp.float32)

    def body(i, _):
        r = i // (HID // 16)
        k = i % (HID // 16)
        rows[r, pl.ds(k * 16, 16)] = z
        return 0

    lax.fori_loop(0, CH * (HID // 16), body, 0)


def _deg_body(src_hbm, dst_hbm, dout_hbm, din_hbm, srcv, dstv, onesv, zbuf,
              dout_sh, din_sh, sem, fsem):
    c = lax.axis_index("c")
    s = lax.axis_index("s")
    wid = s * NC + c

    i1 = pltpu.async_copy(src_hbm.at[wid], srcv, sem)
    i2 = pltpu.async_copy(dst_hbm.at[wid], dstv, sem)

    one = jnp.full((16,), 1.0, jnp.float32)
    z = jnp.zeros((16,), jnp.float32)

    def fill_ones(i, _):
        onesv[pl.ds(i * 16, 16)] = one
        return 0

    lax.fori_loop(0, CH // 16, fill_ones, 0)

    def fill_z(i, _):
        zbuf[pl.ds(i * 16, 16)] = z
        return 0

    lax.fori_loop(0, RPT // 16, fill_z, 0)

    pltpu.sync_copy(zbuf, dout_sh.at[pl.ds(s * RPT, RPT)])
    pltpu.sync_copy(zbuf, din_sh.at[pl.ds(s * RPT, RPT)])
    plsc.subcore_barrier()
    i1.wait()
    i2.wait()

    def add_src(j, _):
        pltpu.async_copy(onesv, dout_sh.at[srcv.at[j]], fsem, add=True)
        return 0

    lax.fori_loop(0, CPT, add_src, 0)

    def add_dst(j, _):
        pltpu.async_copy(onesv, din_sh.at[dstv.at[j]], fsem, add=True)
        return 0

    lax.fori_loop(0, CPT, add_dst, 0)

    def drain(j, _):
        pltpu.make_async_copy(onesv, dout_sh.at[srcv.at[0]], fsem).wait()
        return 0

    lax.fori_loop(0, 2 * CPT, drain, 0)

    plsc.subcore_barrier()

    @pl.when(s == 0)
    def _():
        pltpu.sync_copy(dout_sh, dout_hbm.at[pl.ds(c * NPAD, NPAD)])
        pltpu.sync_copy(din_sh, din_hbm.at[pl.ds(c * NPAD, NPAD)])


def _sc_degrees(srcp, dstp):
    call = pl.kernel(
        _deg_body,
        out_type=[jax.ShapeDtypeStruct((NC * NPAD,), jnp.float32),
                  jax.ShapeDtypeStruct((NC * NPAD,), jnp.float32)],
        mesh=_mesh(),
        scratch_types=[
            pltpu.VMEM((CPT, CH), jnp.int32),
            pltpu.VMEM((CPT, CH), jnp.int32),
            pltpu.VMEM((CH,), jnp.float32),
            pltpu.VMEM((RPT,), jnp.float32),
            pltpu.VMEM_SHARED((NPAD,), jnp.float32),
            pltpu.VMEM_SHARED((NPAD,), jnp.float32),
            pltpu.SemaphoreType.DMA,
            pltpu.SemaphoreType.DMA,
        ],
    )
    return call(srcp, dstp)


def _mp_body(src_hbm, dst_hbm, hs_hbm, out_hbm, srcv, dstv, rows, agg_sh,
             sem, gsem, ssem):
    c = lax.axis_index("c")
    s = lax.axis_index("s")
    wid = s * NC + c

    i1 = pltpu.async_copy(src_hbm.at[wid].at[0], srcv, sem)
    i2 = pltpu.async_copy(dst_hbm.at[wid].at[0], dstv, sem)
    _zero_rows(rows.at[0])
    for k in range(RPT // CH):
        pltpu.sync_copy(rows.at[0], agg_sh.at[pl.ds(s * RPT + k * CH, CH)])
    plsc.subcore_barrier()
    i1.wait()
    i2.wait()

    def wait_g(b):
        pltpu.make_async_copy(hs_hbm.at[srcv.at[0]], rows.at[b],
                              gsem.at[b]).wait()

    def wait_s(b):
        pltpu.make_async_copy(rows.at[b], agg_sh.at[dstv.at[0]],
                              ssem.at[b]).wait()

    for h in range(NH):
        if h > 0:
            pltpu.async_copy(src_hbm.at[wid].at[h], srcv, sem).wait()
            pltpu.async_copy(dst_hbm.at[wid].at[h], dstv, sem).wait()

        pltpu.async_copy(hs_hbm.at[srcv.at[0]], rows.at[0], gsem.at[0])

        def rstep(r, _):
            j = r * 2

            @pl.when(r >= 1)
            def _():
                wait_s(1)

            pltpu.async_copy(hs_hbm.at[srcv.at[j + 1]], rows.at[1],
                             gsem.at[1])
            wait_g(0)
            pltpu.async_copy(rows.at[0], agg_sh.at[dstv.at[j]], ssem.at[0],
                             add=True)
            wait_s(0)

            @pl.when(r < CPH // 2 - 1)
            def _():
                pltpu.async_copy(hs_hbm.at[srcv.at[j + 2]], rows.at[0],
                                 gsem.at[0])

            wait_g(1)
            pltpu.async_copy(rows.at[1], agg_sh.at[dstv.at[j + 1]],
                             ssem.at[1], add=True)
            return 0

        lax.fori_loop(0, CPH // 2, rstep, 0)
        wait_s(1)

    plsc.subcore_barrier()

    for k in range(RPT // CH):
        off = s * RPT + k * CH
        pltpu.sync_copy(agg_sh.at[pl.ds(off, CH)],
                        out_hbm.at[pl.ds(c * NPAD + off, CH)])


def _sc_message_pass(srcp, dstp, hs):
    call = pl.kernel(
        _mp_body,
        out_type=jax.ShapeDtypeStruct((NC * NPAD, HID), jnp.float32),
        mesh=_mesh(),
        scratch_types=[
            pltpu.VMEM((CPH, CH), jnp.int32),
            pltpu.VMEM((CPH, CH), jnp.int32),
            pltpu.VMEM((NBUF, CH, HID), jnp.float32),
            pltpu.VMEM_SHARED((NPAD, HID), jnp.float32),
            pltpu.SemaphoreType.DMA,
            pltpu.SemaphoreType.DMA((NBUF,)),
            pltpu.SemaphoreType.DMA((NBUF,)),
        ],
    )
    return call(srcp.reshape(NW, NH, CPH, CH), dstp.reshape(NW, NH, CPH, CH),
                hs)


def _prep_body(pos_ref, wpos_ref, bpos_ref, dout_ref, din_ref,
               hf_ref, hs_ref, ns_ref, nd_ref):
    hf = jnp.dot(pos_ref[...], wpos_ref[...],
                 preferred_element_type=jnp.float32) + bpos_ref[...]
    do = dout_ref[0] + dout_ref[1]
    di = din_ref[0] + din_ref[1]
    ns = jnp.where(do > 0, lax.rsqrt(jnp.maximum(do, 1.0)), 0.0)
    nd = jnp.where(di > 0, lax.rsqrt(jnp.maximum(di, 1.0)), 0.0)
    hf_ref[...] = hf
    hs_ref[...] = hf * ns
    ns_ref[...] = ns
    nd_ref[...] = nd


def _tc_prep(posp, wpos, bpos, dout, din):
    grid = (NPAD // BR,)
    return pl.pallas_call(
        _prep_body,
        grid=grid,
        in_specs=[
            pl.BlockSpec((BR, POS), lambda i: (i, 0)),
            pl.BlockSpec((POS, HID), lambda i: (0, 0)),
            pl.BlockSpec((1, HID), lambda i: (0, 0)),
            pl.BlockSpec((NC, BR, 1), lambda i: (0, i, 0)),
            pl.BlockSpec((NC, BR, 1), lambda i: (0, i, 0)),
        ],
        out_specs=[
            pl.BlockSpec((BR, HID), lambda i: (i, 0)),
            pl.BlockSpec((BR, HID), lambda i: (i, 0)),
            pl.BlockSpec((BR, 1), lambda i: (i, 0)),
            pl.BlockSpec((BR, 1), lambda i: (i, 0)),
        ],
        out_shape=[
            jax.ShapeDtypeStruct((NPAD, HID), jnp.float32),
            jax.ShapeDtypeStruct((NPAD, HID), jnp.float32),
            jax.ShapeDtypeStruct((NPAD, 1), jnp.float32),
            jax.ShapeDtypeStruct((NPAD, 1), jnp.float32),
        ],
    )(posp, wpos, bpos, dout, din)


def _layer_body(agg_ref, nd_ref, ns_ref, hf_ref, w_ref, b_ref,
                hf_out, hs_out):
    agg = (agg_ref[0] + agg_ref[1]) * nd_ref[...]
    out = jnp.dot(agg, w_ref[...], preferred_element_type=jnp.float32)
    out = jnp.maximum(out + b_ref[...], 0.0)
    hfn = hf_ref[...] + out
    hf_out[...] = hfn
    hs_out[...] = hfn * ns_ref[...]


def _tc_layer(aggp, nd, ns, hf, w, b):
    grid = (NPAD // BR,)
    return pl.pallas_call(
        _layer_body,
        grid=grid,
        in_specs=[
            pl.BlockSpec((NC, BR, HID), lambda i: (0, i, 0)),
            pl.BlockSpec((BR, 1), lambda i: (i, 0)),
            pl.BlockSpec((BR, 1), lambda i: (i, 0)),
            pl.BlockSpec((BR, HID), lambda i: (i, 0)),
            pl.BlockSpec((HID, HID), lambda i: (0, 0)),
            pl.BlockSpec((1, HID), lambda i: (0, 0)),
        ],
        out_specs=[
            pl.BlockSpec((BR, HID), lambda i: (i, 0)),
            pl.BlockSpec((BR, HID), lambda i: (i, 0)),
        ],
        out_shape=[
            jax.ShapeDtypeStruct((NPAD, HID), jnp.float32),
            jax.ShapeDtypeStruct((NPAD, HID), jnp.float32),
        ],
    )(aggp, nd, ns, hf, w, b)


def _layer4_body(agg_ref, nd_ref, hf_ref, w_ref, b_ref, w0_ref, b0_ref,
                 w1_ref, b1_ref, w2_ref, b2_ref, y_ref, acc_ref):
    i = pl.program_id(0)
    agg = (agg_ref[0] + agg_ref[1]) * nd_ref[...]
    out = jnp.dot(agg, w_ref[...], preferred_element_type=jnp.float32)
    out = jnp.maximum(out + b_ref[...], 0.0)
    hfn = hf_ref[...] + out
    rid = lax.broadcasted_iota(jnp.int32, (BR, 1), 0) + i * BR
    bsum = jnp.sum(jnp.where(rid < N, hfn, 0.0), axis=0, keepdims=True)

    @pl.when(i == 0)
    def _():
        acc_ref[...] = bsum

    @pl.when(i > 0)
    def _():
        acc_ref[...] += bsum

    @pl.when(i == NPAD // BR - 1)
    def _():
        hg = acc_ref[...] * (1.0 / N)
        y = jnp.maximum(jnp.dot(hg, w0_ref[...],
                                preferred_element_type=jnp.float32)
                        + b0_ref[...], 0.0)
        y = jnp.maximum(jnp.dot(y, w1_ref[...],
                                preferred_element_type=jnp.float32)
                        + b1_ref[...], 0.0)
        y = jnp.dot(y, w2_ref[...], preferred_element_type=jnp.float32) \
            + b2_ref[...]
        y_ref[...] = y


def _tc_layer4(aggp, nd, hf, w, b, w0, b0, w1, b1, w2, b2):
    grid = (NPAD // BR,)
    const = pl.BlockSpec((1, HID), lambda i: (0, 0))
    mat = pl.BlockSpec((HID, HID), lambda i: (0, 0))
    return pl.pallas_call(
        _layer4_body,
        grid=grid,
        in_specs=[
            pl.BlockSpec((NC, BR, HID), lambda i: (0, i, 0)),
            pl.BlockSpec((BR, 1), lambda i: (i, 0)),
            pl.BlockSpec((BR, HID), lambda i: (i, 0)),
            mat, const, mat, const, mat, const, mat, const,
        ],
        out_specs=pl.BlockSpec((1, HID), lambda i: (0, 0)),
        out_shape=jax.ShapeDtypeStruct((1, HID), jnp.float32),
        scratch_shapes=[pltpu.VMEM((1, HID), jnp.float32)],
    )(aggp, nd, hf, w, b, w0, b0, w1, b1, w2, b2)


def _pad_mat(w, rows, cols):
    out = jnp.zeros((rows, cols), w.dtype)
    return out.at[: w.shape[0], : w.shape[1]].set(w)


def kernel(edge_index, h, e, pos_enc, params):
    del h, e
    src = edge_index[0]
    dst = edge_index[1]

    npad_e = EPAD - E
    pad_idx = N + (jnp.arange(npad_e, dtype=jnp.int32) % (NPAD - N))
    srcp = jnp.concatenate([src, pad_idx]).reshape(NW, CPT, CH)
    dstp = jnp.concatenate([dst, pad_idx]).reshape(NW, CPT, CH)

    posp = jnp.zeros((NPAD, POS), jnp.float32).at[:N].set(pos_enc)

    dout, din = _sc_degrees(srcp, dstp)
    dout = dout.reshape(NC, NPAD, 1)
    din = din.reshape(NC, NPAD, 1)

    hf, hs, ns, nd = _tc_prep(posp, params["W_pos"],
                              params["b_pos"].reshape(1, HID), dout, din)

    inv_bn = 1.0 / np.sqrt(1.0 + BN_EPS)

    def fold(l):
        g = params["gamma"][l] * inv_bn
        w = params["W"][l] * g[None, :]
        b = (params["b"][l] * g + params["beta"][l]).reshape(1, HID)
        return w, b

    for l in range(NLAYER - 1):
        w, b = fold(l)
        aggp = _sc_message_pass(srcp, dstp, hs).reshape(NC, NPAD, HID)
        hf, hs = _tc_layer(aggp, nd, ns, hf, w, b)

    w0 = _pad_mat(params["Wr"][0], HID, HID)
    b0 = _pad_mat(params["br"][0].reshape(1, -1), 1, HID)
    w1 = _pad_mat(params["Wr"][1], HID, HID)
    b1 = _pad_mat(params["br"][1].reshape(1, -1), 1, HID)
    w2 = _pad_mat(params["Wr"][2], HID, HID)
    b2 = _pad_mat(params["br"][2].reshape(1, -1), 1, HID)

    w, b = fold(NLAYER - 1)
    aggp = _sc_message_pass(srcp, dstp, hs).reshape(NC, NPAD, HID)
    y = _tc_layer4(aggp, nd, hf, w, b, w0, b0, w1, b1, w2, b2)
    return y[:, :NCLS]
